# Initial kernel scaffold; baseline (speedup 1.0000x reference)
import jax
import jax.numpy as jnp
from jax.experimental import pallas as pl


def kernel(input, target):
    raise NotImplementedError("write your pallas kernel here")


# speedup vs baseline: 58.8243x; 58.8243x over previous
import functools

import jax
import jax.numpy as jnp
from jax.experimental import pallas as pl
from jax.experimental.pallas import tpu as pltpu

_SMOOTH = 1e-05
_LB = 16384


def _stats_kernel(x_ref, t_ref, stats_ref, loss_ref, *, nblocks):
    j = pl.program_id(1)
    x = x_ref[0]
    t = t_ref[0, 0]
    C, LB = x.shape

    m = jnp.max(x, axis=0, keepdims=True)
    s = jnp.sum(jnp.exp(x - m), axis=0, keepdims=True)
    lse = jnp.log(s) + m

    cls = jax.lax.broadcasted_iota(jnp.int32, (C, LB), 0)
    onehot = t == cls

    amax = jnp.min(jnp.where(x == m, cls, C), axis=0, keepdims=True)
    correct = amax == t

    picked = jnp.sum(jnp.where(onehot, x, 0.0), axis=0, keepdims=True)
    logp = picked - lse

    tt = jnp.sum(onehot.astype(jnp.float32), axis=1, keepdims=True)
    tp = jnp.sum(jnp.where(onehot & correct, 1.0, 0.0), axis=1, keepdims=True)
    S = jnp.sum(jnp.where(onehot, logp, 0.0), axis=1, keepdims=True)
    acc = jnp.concatenate([tt, tp, S], axis=1)

    @pl.when(j == 0)
    def _():
        stats_ref[0] = acc

    @pl.when(j != 0)
    def _():
        stats_ref[0] = stats_ref[0] + acc

    @pl.when(j == nblocks - 1)
    def _():
        st = stats_ref[0]
        tt_a = st[:, 0:1]
        tp_a = st[:, 1:2]
        s_a = st[:, 2:3]
        rw = 1.0 - (tp_a + _SMOOTH) / (tt_a + _SMOOTH)
        num = jnp.sum(rw * s_a)
        den = jnp.sum(rw * tt_a)
        loss_ref[...] = (-num / den).reshape(1, 1, 1)


def kernel(input, target):
    N, C = input.shape[0], input.shape[1]
    L = input.shape[2] * input.shape[3]
    x = input.reshape(N, C, L)
    t = target.astype(jnp.int32).reshape(N, L // _LB, 1, _LB)
    nblocks = L // _LB

    stats, loss = pl.pallas_call(
        functools.partial(_stats_kernel, nblocks=nblocks),
        grid=(N, nblocks),
        in_specs=[
            pl.BlockSpec((1, C, _LB), lambda n, j: (n, 0, j)),
            pl.BlockSpec((1, 1, 1, _LB), lambda n, j: (n, j, 0, 0)),
        ],
        out_specs=[
            pl.BlockSpec((1, C, 3), lambda n, j: (n, 0, 0)),
            pl.BlockSpec((1, 1, 1), lambda n, j: (n, 0, 0)),
        ],
        out_shape=[
            jax.ShapeDtypeStruct((N, C, 3), jnp.float32),
            jax.ShapeDtypeStruct((N, 1, 1), jnp.float32),
        ],
        compiler_params=pltpu.CompilerParams(
            dimension_semantics=("parallel", "arbitrary"),
        ),
    )(x, t)
    return loss[:, 0, 0]
